# Initial kernel scaffold; baseline (speedup 1.0000x reference)
import jax
import jax.numpy as jnp
from jax.experimental import pallas as pl


def kernel(x, token_table, pos_table):
    raise NotImplementedError("write your pallas kernel here")


# speedup vs baseline: 1.2786x; 1.2786x over previous
import functools

import jax
import jax.numpy as jnp
from jax import lax
from jax.experimental import pallas as pl
from jax.experimental.pallas import tpu as pltpu
from jax.experimental.pallas import tpu_sc as plsc

_LANES = 16


@functools.partial(jax.jit, static_argnames=("nw", "chunk", "k", "seq"))
def _sc_embed(idx, token_table, pos_table, *, nw, chunk, k, seq):
    total = nw * chunk * k
    hidden = token_table.shape[1]
    rows_w = chunk * k

    mesh = plsc.VectorSubcoreMesh(core_axis_name="c", subcore_axis_name="s")

    @functools.partial(
        pl.kernel,
        out_type=jax.ShapeDtypeStruct((total, hidden), jnp.float32),
        mesh=mesh,
        scratch_types=[
            pltpu.VMEM((k, chunk), jnp.int32),
            pltpu.VMEM((rows_w, hidden), jnp.float32),
            pltpu.VMEM((rows_w, hidden), jnp.float32),
            pltpu.SemaphoreType.DMA,
            pltpu.SemaphoreType.DMA,
        ],
    )
    def body(idx_hbm, tok_hbm, pos_hbm, out_hbm, idx_v, rows_v, pos_v, sem0, sem1):
        wid = lax.axis_index("s") * 2 + lax.axis_index("c")
        base = wid * rows_w
        pos_base = lax.rem(base, seq)

        pltpu.sync_copy(idx_hbm.at[pl.ds(wid * k, k)], idx_v)
        cp0 = pltpu.async_copy(
            tok_hbm.at[idx_v.at[0]], rows_v.at[pl.ds(0, chunk)], sem0
        )
        cp1 = pltpu.async_copy(
            tok_hbm.at[idx_v.at[1]], rows_v.at[pl.ds(chunk, chunk)], sem1
        )
        pltpu.sync_copy(pos_hbm.at[pl.ds(pos_base, rows_w)], pos_v)
        cp0.wait()
        cp1.wait()

        def add_row(r, carry):
            for j in range(hidden // _LANES):
                sl = pl.ds(j * _LANES, _LANES)
                rows_v[r, sl] = rows_v[r, sl] + pos_v[r, sl]
            return carry

        lax.fori_loop(0, rows_w, add_row, 0)

        pltpu.sync_copy(rows_v, out_hbm.at[pl.ds(base, rows_w)])

    return body(idx, token_table, pos_table)


def kernel(x, token_table, pos_table):
    batch, seq = x.shape
    hidden = token_table.shape[1]
    nw, chunk = 32, 128
    total = batch * seq
    k = total // (nw * chunk)
    idx = x.astype(jnp.int32).reshape(nw * k, chunk)
    out = _sc_embed(idx, token_table, pos_table, nw=nw, chunk=chunk, k=k, seq=seq)
    return out.reshape(batch, seq, hidden)


# speedup vs baseline: 1.3902x; 1.0872x over previous
import functools

import jax
import jax.numpy as jnp
from jax import lax
from jax.experimental import pallas as pl
from jax.experimental.pallas import tpu as pltpu
from jax.experimental.pallas import tpu_sc as plsc

_LANES = 16


@functools.partial(jax.jit, static_argnames=("nw", "batch", "seq"))
def _sc_embed(x, token_table, pos_table, *, nw, batch, seq):
    hidden = token_table.shape[1]
    spw = seq // nw
    rows_w = batch * spw
    lanes = hidden // _LANES

    mesh = plsc.VectorSubcoreMesh(core_axis_name="c", subcore_axis_name="s")

    @functools.partial(
        pl.kernel,
        out_type=jax.ShapeDtypeStruct((batch * seq, hidden), jnp.float32),
        mesh=mesh,
        scratch_types=[
            pltpu.VMEM((rows_w,), jnp.int32),
            pltpu.VMEM((rows_w, hidden), jnp.float32),
            pltpu.VMEM((spw, hidden), jnp.float32),
            [pltpu.SemaphoreType.DMA] * 4,
            [pltpu.SemaphoreType.DMA] * 4,
            pltpu.SemaphoreType.DMA,
            pltpu.SemaphoreType.DMA,
        ],
    )
    def body(x_hbm, tok_hbm, pos_hbm, out_hbm, idx_v, rows_v, pos_v,
             isems, gsems, psem, wsem):
        wid = lax.axis_index("s") * 2 + lax.axis_index("c")
        s0 = wid * spw

        icps = [
            pltpu.async_copy(
                x_hbm.at[pl.ds(b * seq + s0, spw)],
                idx_v.at[pl.ds(b * spw, spw)],
                isems[b],
            )
            for b in range(batch)
        ]
        pcp = pltpu.async_copy(pos_hbm.at[pl.ds(s0, spw)], pos_v, psem)

        gcps = []
        for b in range(batch):
            icps[b].wait()
            gcps.append(
                pltpu.async_copy(
                    tok_hbm.at[idx_v.at[pl.ds(b * spw, spw)]],
                    rows_v.at[pl.ds(b * spw, spw)],
                    gsems[b],
                )
            )
        pcp.wait()

        wcps = []
        for b in range(batch):
            gcps[b].wait()

            def row_body(r, carry, _b=b):
                for j in range(lanes):
                    sl = pl.ds(j * _LANES, _LANES)
                    plsc.addupdate(rows_v.at[_b * spw + r, sl], pos_v[r, sl])
                return carry

            lax.fori_loop(0, spw, row_body, 0)
            wcps.append(
                pltpu.async_copy(
                    rows_v.at[pl.ds(b * spw, spw)],
                    out_hbm.at[pl.ds(b * seq + s0, spw)],
                    wsem,
                )
            )
        for cp in wcps:
            cp.wait()

    return body(x, token_table, pos_table)


def kernel(x, token_table, pos_table):
    batch, seq = x.shape
    hidden = token_table.shape[1]
    out = _sc_embed(
        x.astype(jnp.int32).reshape(batch * seq), token_table, pos_table,
        nw=32, batch=batch, seq=seq,
    )
    return out.reshape(batch, seq, hidden)
